# baseline (speedup 1.0000x reference)
import functools

import jax
import jax.numpy as jnp
from jax import lax
from jax.experimental import pallas as pl
from jax.experimental.pallas import tpu as pltpu
from jax.experimental.pallas import tpu_sc as plsc


def _project_body(dep_ref, sid_ref, eid_ref, w_ref, o_ref):
    o_ref[pl.ds(0, 288)] = jnp.dot(
        dep_ref[...], w_ref[0], preferred_element_type=jnp.float32
    )
    o_ref[pl.ds(288, 257)] = jnp.dot(
        sid_ref[...], w_ref[1], preferred_element_type=jnp.float32
    )
    o_ref[pl.ds(576, 257)] = jnp.dot(
        eid_ref[...], w_ref[2], preferred_element_type=jnp.float32
    )


def _combine_body(g_ref, attr_ref, w2_ref, wpad_ref, b1_ref, bsum_ref, o_ref):
    h = jnp.maximum(g_ref[...] + b1_ref[...], 0.0)
    o_ref[...] = (
        jnp.dot(h, w2_ref[...], preferred_element_type=jnp.float32)
        + jnp.dot(attr_ref[...], wpad_ref[...], preferred_element_type=jnp.float32)
        + bsum_ref[...]
    )


def _combine_body_alias(g_ref, attr_ref, w2_ref, wpad_ref, b1_ref, bsum_ref, prev_ref, o_ref):
    del prev_ref
    _combine_body(g_ref, attr_ref, w2_ref, wpad_ref, b1_ref, bsum_ref, o_ref)


def _sc_gather_sum(P_flat, idx3, B, E):
    info = plsc.get_sparse_core_info()
    NC, NS = info.num_cores, info.num_subcores
    NW = NC * NS
    per_w = B // NW
    CHUNK = 128
    n_chunks = per_w // CHUNK
    mesh = plsc.VectorSubcoreMesh(core_axis_name="c", subcore_axis_name="s")

    VP = P_flat.shape[0]

    @functools.partial(
        pl.kernel,
        mesh=mesh,
        out_type=jax.ShapeDtypeStruct((B, E), jnp.float32),
        scratch_types=[
            pltpu.VMEM((3, n_chunks, CHUNK), jnp.int32),
            pltpu.VMEM((per_w, E), jnp.float32),
            pltpu.VMEM_SHARED((VP, E), jnp.float32),
            pltpu.SemaphoreType.DMA((n_chunks,)),
            pltpu.SemaphoreType.DMA,
        ],
    )
    def gather_k(table_hbm, idx_hbm, out_hbm, idxv, acc, tab_sp, dsem, asem):
        wid = lax.axis_index("s") * NC + lax.axis_index("c")
        base = wid * per_w
        @pl.when(lax.axis_index("s") == 0)
        def _():
            pltpu.sync_copy(table_hbm, tab_sp)

        pltpu.sync_copy(idx_hbm.at[wid], idxv)
        plsc.subcore_barrier()
        first = [
            pltpu.async_copy(
                tab_sp.at[idxv.at[0, c]], acc.at[pl.ds(c * CHUNK, CHUNK)], dsem.at[c]
            )
            for c in range(n_chunks)
        ]
        adds = []
        for c in range(n_chunks):
            first[c].wait()
            for t in (1, 2):
                adds.append(
                    pltpu.async_copy(
                        tab_sp.at[idxv.at[t, c]],
                        acc.at[pl.ds(c * CHUNK, CHUNK)],
                        asem,
                        add=True,
                    )
                )
        for cp in adds:
            cp.wait()
        pltpu.sync_copy(acc, out_hbm.at[pl.ds(base, per_w)])

    return gather_k(P_flat, idx3)


def kernel(attr, W_wide, b_wide, dep_table, sid_table, eid_table, W1, b1, W2, b2):
    B = attr.shape[0]
    H = dep_table.shape[1]
    E = W1.shape[1]
    V = 288

    W1r = W1.reshape(3, H, E)

    NW = 32
    CHUNK = 128
    Bh = B // 2
    nck = Bh // (NW * CHUNK)
    dep = attr[:, 0].astype(jnp.int32).reshape(2, NW, nck, CHUNK)
    sid = attr[:, 6].astype(jnp.int32).reshape(2, NW, nck, CHUNK)
    eid = attr[:, 7].astype(jnp.int32).reshape(2, NW, nck, CHUNK)
    idx3 = jnp.stack([dep, V + sid, 2 * V + eid], axis=2)

    W_pad = jnp.zeros((8, E), jnp.float32).at[1:6].set(W_wide)
    b1_2d = b1.reshape(1, E)
    bsum = (b_wide + b2).reshape(1, E)

    P_flat = pl.pallas_call(
        _project_body,
        in_specs=[
            pl.BlockSpec(dep_table.shape, lambda: (0, 0)),
            pl.BlockSpec(sid_table.shape, lambda: (0, 0)),
            pl.BlockSpec(eid_table.shape, lambda: (0, 0)),
            pl.BlockSpec((3, H, E), lambda: (0, 0, 0)),
        ],
        out_specs=pl.BlockSpec((3 * V, E), lambda: (0, 0)),
        out_shape=jax.ShapeDtypeStruct((3 * V, E), jnp.float32),
    )(dep_table, sid_table, eid_table, W1r)

    G1 = _sc_gather_sum(P_flat, idx3[0], Bh, E)
    G2 = _sc_gather_sum(P_flat, idx3[1], Bh, E)

    R = 2048
    nblk = Bh // R
    wspecs = [
        pl.BlockSpec((E, E), lambda i: (0, 0)),
        pl.BlockSpec((8, E), lambda i: (0, 0)),
        pl.BlockSpec((1, E), lambda i: (0, 0)),
        pl.BlockSpec((1, E), lambda i: (0, 0)),
    ]
    out1 = pl.pallas_call(
        _combine_body,
        grid=(nblk,),
        in_specs=[
            pl.BlockSpec((R, E), lambda i: (i, 0)),
            pl.BlockSpec((R, 8), lambda i: (i, 0)),
        ]
        + wspecs,
        out_specs=pl.BlockSpec((R, E), lambda i: (i, 0)),
        out_shape=jax.ShapeDtypeStruct((B, E), jnp.float32),
    )(G1, attr, W2, W_pad, b1_2d, bsum)
    out = pl.pallas_call(
        _combine_body_alias,
        grid=(nblk,),
        in_specs=[
            pl.BlockSpec((R, E), lambda i: (i, 0)),
            pl.BlockSpec((R, 8), lambda i: (i + nblk, 0)),
        ]
        + wspecs
        + [pl.BlockSpec(memory_space=pl.ANY)],
        out_specs=pl.BlockSpec((R, E), lambda i: (i + nblk, 0)),
        out_shape=jax.ShapeDtypeStruct((B, E), jnp.float32),
        input_output_aliases={6: 0},
    )(G2, attr, W2, W_pad, b1_2d, bsum, out1)
    return out


# speedup vs baseline: 1.0938x; 1.0938x over previous
import functools

import jax
import jax.numpy as jnp
from jax import lax
from jax.experimental import pallas as pl
from jax.experimental.pallas import tpu as pltpu
from jax.experimental.pallas import tpu_sc as plsc


def _project_body(dep_ref, sid_ref, eid_ref, w_ref, o_ref):
    o_ref[pl.ds(0, 288)] = jnp.dot(
        dep_ref[...], w_ref[0], preferred_element_type=jnp.float32
    )
    o_ref[pl.ds(288, 257)] = jnp.dot(
        sid_ref[...], w_ref[1], preferred_element_type=jnp.float32
    )
    o_ref[pl.ds(576, 257)] = jnp.dot(
        eid_ref[...], w_ref[2], preferred_element_type=jnp.float32
    )


def _combine_body(g_ref, attr_ref, w2_ref, wpad_ref, b1_ref, bsum_ref, o_ref):
    h = jnp.maximum(g_ref[...] + b1_ref[...], 0.0)
    o_ref[...] = (
        jnp.dot(h, w2_ref[...], preferred_element_type=jnp.float32)
        + jnp.dot(attr_ref[...], wpad_ref[...], preferred_element_type=jnp.float32)
        + bsum_ref[...]
    )


def _sc_gather_sum(P_flat, idx3, B, E):
    info = plsc.get_sparse_core_info()
    NC, NS = info.num_cores, info.num_subcores
    NW = NC * NS
    per_w = B // NW
    CHUNK = 128
    n_chunks = per_w // CHUNK
    VP = P_flat.shape[0]
    mesh = plsc.VectorSubcoreMesh(core_axis_name="c", subcore_axis_name="s")

    @functools.partial(
        pl.kernel,
        mesh=mesh,
        out_type=jax.ShapeDtypeStruct((B, E), jnp.float32),
        scratch_types=[
            pltpu.VMEM((3, n_chunks, CHUNK), jnp.int32),
            pltpu.VMEM((per_w, E), jnp.float32),
            pltpu.VMEM_SHARED((VP, E), jnp.float32),
            pltpu.SemaphoreType.DMA((n_chunks,)),
            pltpu.SemaphoreType.DMA,
        ],
    )
    def gather_k(table_hbm, idx_hbm, out_hbm, idxv, acc, tab_sp, dsem, asem):
        wid = lax.axis_index("s") * NC + lax.axis_index("c")
        base = wid * per_w
        @pl.when(lax.axis_index("s") == 0)
        def _():
            pltpu.sync_copy(table_hbm, tab_sp)

        pltpu.sync_copy(idx_hbm.at[wid], idxv)
        plsc.subcore_barrier()
        first = [
            pltpu.async_copy(
                tab_sp.at[idxv.at[0, c]], acc.at[pl.ds(c * CHUNK, CHUNK)], dsem.at[c]
            )
            for c in range(n_chunks)
        ]
        adds = []
        for c in range(n_chunks):
            first[c].wait()
            for t in (1, 2):
                adds.append(
                    pltpu.async_copy(
                        tab_sp.at[idxv.at[t, c]],
                        acc.at[pl.ds(c * CHUNK, CHUNK)],
                        asem,
                        add=True,
                    )
                )
        for cp in adds:
            cp.wait()
        pltpu.sync_copy(acc, out_hbm.at[pl.ds(base, per_w)])

    return gather_k(P_flat, idx3)


def kernel(attr, W_wide, b_wide, dep_table, sid_table, eid_table, W1, b1, W2, b2):
    B = attr.shape[0]
    H = dep_table.shape[1]
    E = W1.shape[1]
    V = 288

    W1r = W1.reshape(3, H, E)

    NW = 32
    CHUNK = 128
    nck = B // (NW * CHUNK)
    dep = attr[:, 0].astype(jnp.int32).reshape(NW, nck, CHUNK)
    sid = attr[:, 6].astype(jnp.int32).reshape(NW, nck, CHUNK)
    eid = attr[:, 7].astype(jnp.int32).reshape(NW, nck, CHUNK)
    idx3 = jnp.stack([dep, V + sid, 2 * V + eid], axis=1)

    W_pad = jnp.zeros((8, E), jnp.float32).at[1:6].set(W_wide)
    b1_2d = b1.reshape(1, E)
    bsum = (b_wide + b2).reshape(1, E)

    P_flat = pl.pallas_call(
        _project_body,
        in_specs=[
            pl.BlockSpec(dep_table.shape, lambda: (0, 0)),
            pl.BlockSpec(sid_table.shape, lambda: (0, 0)),
            pl.BlockSpec(eid_table.shape, lambda: (0, 0)),
            pl.BlockSpec((3, H, E), lambda: (0, 0, 0)),
        ],
        out_specs=pl.BlockSpec((3 * V, E), lambda: (0, 0)),
        out_shape=jax.ShapeDtypeStruct((3 * V, E), jnp.float32),
    )(dep_table, sid_table, eid_table, W1r)

    G = _sc_gather_sum(P_flat, idx3, B, E)

    R = 4096
    out = pl.pallas_call(
        _combine_body,
        grid=(B // R,),
        in_specs=[
            pl.BlockSpec((R, E), lambda i: (i, 0)),
            pl.BlockSpec((R, 8), lambda i: (i, 0)),
            pl.BlockSpec((E, E), lambda i: (0, 0)),
            pl.BlockSpec((8, E), lambda i: (0, 0)),
            pl.BlockSpec((1, E), lambda i: (0, 0)),
            pl.BlockSpec((1, E), lambda i: (0, 0)),
        ],
        out_specs=pl.BlockSpec((R, E), lambda i: (i, 0)),
        out_shape=jax.ShapeDtypeStruct((B, E), jnp.float32),
    )(G, attr, W2, W_pad, b1_2d, bsum)
    return out
